# baseline (speedup 1.0000x reference)
import math

import jax
import jax.numpy as jnp
from jax.experimental import pallas as pl
from jax.experimental.pallas import tpu as pltpu

_HEADS = 16
_PAIRS = 8
_RT = 4


def _body(x_ref, wq_ref, wk_ref, wv_ref, wo_ref, misc_ref, out_ref,
          q3_ref, k3_ref, v3_ref, c3_ref):
    p = pl.program_id(0)
    T = x_ref.shape[0]
    D = x_ref.shape[1]
    rt = T // _RT
    cdims = (((1,), (1,)), ((), ()))

    @pl.when(p == 0)
    def _conv_proj():
        zero = jnp.zeros((1, D), jnp.float32)
        for i in range(_RT):
            lo = i * rt
            xv = x_ref[lo:lo + rt, :]
            if i == 0:
                xm = jnp.concatenate([zero, x_ref[0:rt - 1, :]], axis=0)
            else:
                xm = x_ref[lo - 1:lo + rt - 1, :]
            if i == _RT - 1:
                xp = jnp.concatenate([x_ref[lo + 1:T, :], zero], axis=0)
            else:
                xp = x_ref[lo + 1:lo + rt + 1, :]
            for plane, w_ref, base, brow in ((q3_ref, wq_ref, 0, 10),
                                             (k3_ref, wk_ref, 3, 11),
                                             (v3_ref, wv_ref, 6, 12)):
                w0 = misc_ref[base, :][None, :]
                w1 = misc_ref[base + 1, :][None, :]
                w2 = misc_ref[base + 2, :][None, :]
                y = (xm * w0 + xv * w1 + xp * w2).astype(jnp.bfloat16)
                t = (jax.lax.dot_general(
                        y, w_ref[...], cdims,
                        preferred_element_type=jnp.float32)
                     + misc_ref[brow, :][None, :]).astype(jnp.bfloat16)
                plane[:, lo:lo + rt, :] = (
                    t.reshape(rt, _PAIRS, 128).swapaxes(0, 1))

    kp = k3_ref[p]
    vp = v3_ref[p]
    qp = q3_ref[p]
    ka = kp[:, 0:64]
    kb = kp[:, 64:128]
    va = vp[:, 0:64]
    vb = vp[:, 64:128]

    for i in range(_RT):
        lo = i * rt
        ctx2 = []
        for k_h, v_h, c0 in ((ka, va, 0), (kb, vb, 64)):
            q = qp[lo:lo + rt, c0:c0 + 64]
            scores = jax.lax.dot_general(q, k_h, cdims,
                                         preferred_element_type=jnp.float32)
            m = jnp.max(scores, axis=1, keepdims=True)
            pr = jnp.exp(scores - m).astype(jnp.bfloat16)
            s = jnp.sum(pr.astype(jnp.float32), axis=1, keepdims=True)
            ctx = (jax.lax.dot_general(pr, v_h, (((1,), (0,)), ((), ())),
                                       preferred_element_type=jnp.float32)
                   / s).astype(jnp.bfloat16)
            ctx2.append(ctx)
        c3_ref[p, lo:lo + rt, :] = jnp.concatenate(ctx2, axis=1)

    @pl.when(p == _PAIRS - 1)
    def _out():
        ob = misc_ref[9, :][None, :]
        for i in range(_RT):
            lo = i * rt
            cc = jnp.concatenate([c3_ref[pp, lo:lo + rt, :]
                                  for pp in range(_PAIRS)], axis=1)
            out_ref[lo:lo + rt, :] = jax.lax.dot_general(
                cc, wo_ref[...], cdims,
                preferred_element_type=jnp.float32) + ob


def kernel(x, q_dw_w, q_dw_b, q_bn_g, q_bn_b, q_pw_w, q_pw_b,
           k_dw_w, k_dw_b, k_bn_g, k_bn_b, k_pw_w, k_pw_b,
           v_dw_w, v_dw_b, v_bn_g, v_bn_b, v_pw_w, v_pw_b,
           out_w, out_b):
    B, T, D = x.shape
    dk = D // _HEADS
    x2d = x[0]
    inv = 1.0 / math.sqrt(1.0 + 1e-5)

    def fold(pw_w, bn_g, bn_b, dw_b, pw_b):
        a = bn_g * inv
        w_eff = pw_w[:, :, 0] * a[None, :]
        b_eff = pw_w[:, :, 0] @ (dw_b * a + bn_b) + pw_b
        return w_eff, b_eff

    wq, bq = fold(q_pw_w, q_bn_g, q_bn_b, q_dw_b, q_pw_b)
    scale = 1.0 / math.sqrt(dk)
    wq = wq * scale
    bq = bq * scale
    wk, bk = fold(k_pw_w, k_bn_g, k_bn_b, k_dw_b, k_pw_b)
    wv, bv = fold(v_pw_w, v_bn_g, v_bn_b, v_dw_b, v_pw_b)

    rows = [q_dw_w[:, 0, 0], q_dw_w[:, 0, 1], q_dw_w[:, 0, 2],
            k_dw_w[:, 0, 0], k_dw_w[:, 0, 1], k_dw_w[:, 0, 2],
            v_dw_w[:, 0, 0], v_dw_w[:, 0, 1], v_dw_w[:, 0, 2],
            out_b, bq, bk, bv]
    misc = jnp.stack(rows, axis=0)

    out2d = pl.pallas_call(
        _body,
        grid=(_PAIRS,),
        in_specs=[
            pl.BlockSpec((T, D), lambda p: (0, 0)),
            pl.BlockSpec((D, D), lambda p: (0, 0)),
            pl.BlockSpec((D, D), lambda p: (0, 0)),
            pl.BlockSpec((D, D), lambda p: (0, 0)),
            pl.BlockSpec((D, D), lambda p: (0, 0)),
            pl.BlockSpec((13, D), lambda p: (0, 0)),
        ],
        out_specs=pl.BlockSpec((T, D), lambda p: (0, 0)),
        out_shape=jax.ShapeDtypeStruct((T, D), jnp.float32),
        scratch_shapes=[pltpu.VMEM((_PAIRS, T, 128), jnp.bfloat16)] * 4,
        compiler_params=pltpu.CompilerParams(
            dimension_semantics=("arbitrary",)),
    )(x2d, wq.astype(jnp.bfloat16), wk.astype(jnp.bfloat16),
      wv.astype(jnp.bfloat16), out_w.astype(jnp.bfloat16), misc)

    return out2d[None, :, :]
